# Initial kernel scaffold; baseline (speedup 1.0000x reference)
import jax
import jax.numpy as jnp
from jax.experimental import pallas as pl


def kernel(prob, label):
    raise NotImplementedError("write your pallas kernel here")


# speedup vs baseline: 7.9286x; 7.9286x over previous
import functools

import jax
import jax.numpy as jnp
from jax import lax
from jax.experimental import pallas as pl
from jax.experimental.pallas import tpu as pltpu
from jax.experimental.pallas import tpu_sc as plsc

_BATCH = 16384
_NCLS = 392
_NC = 2
_NS = 16
_NW = _NC * _NS
_L = 16
_ROWS_PER_W = _BATCH // _NW
_GROUPS = _ROWS_PER_W // _L

_mesh = plsc.VectorSubcoreMesh(core_axis_name="c", subcore_axis_name="s")


@functools.partial(
    pl.kernel,
    mesh=_mesh,
    out_type=jax.ShapeDtypeStruct((_NW, _L), jnp.float32),
    scratch_types=[
        pltpu.VMEM((_L * _NCLS,), jnp.float32),
        pltpu.VMEM((_ROWS_PER_W,), jnp.int32),
        pltpu.VMEM((_L,), jnp.float32),
    ],
    compiler_params=pltpu.CompilerParams(
        use_tc_tiling_on_sc=False, needs_layout_passes=False),
)
def _iou_partials(prob_hbm, label_hbm, out_hbm, buf, lab_v, acc_v):
    wid = lax.axis_index("s") * _NC + lax.axis_index("c")
    row0 = wid * _ROWS_PER_W
    pltpu.sync_copy(label_hbm.at[pl.ds(row0, _ROWS_PER_W)], lab_v)

    iota = lax.iota(jnp.int32, _L)
    row_base = iota * _NCLS
    neg_inf = jnp.full((_L,), -jnp.inf, jnp.float32)

    def group_body(g, acc):
        pltpu.sync_copy(
            prob_hbm.at[pl.ds((row0 + g * _L) * _NCLS, _L * _NCLS)], buf)

        def j_body(j, ms):
            m0, m1, m2, m3, m4, m5 = ms
            x = plsc.load_gather(buf, [row_base + j])
            n0 = jnp.maximum(m0, x)
            c = jnp.minimum(m0, x)
            n1 = jnp.maximum(m1, c)
            c = jnp.minimum(m1, c)
            n2 = jnp.maximum(m2, c)
            c = jnp.minimum(m2, c)
            n3 = jnp.maximum(m3, c)
            c = jnp.minimum(m3, c)
            n4 = jnp.maximum(m4, c)
            c = jnp.minimum(m4, c)
            n5 = jnp.maximum(m5, c)
            return (n0, n1, n2, n3, n4, n5)

        m0, m1, m2, m3, m4, m5 = lax.fori_loop(
            0, _NCLS, j_body, (neg_inf,) * 6, unroll=8)

        t = m5
        cnt = ((m0 > t).astype(jnp.float32) + (m1 > t).astype(jnp.float32)
               + (m2 > t).astype(jnp.float32) + (m3 > t).astype(jnp.float32)
               + (m4 > t).astype(jnp.float32))
        lab16 = plsc.load_gather(lab_v, [g * _L + iota])
        labval = plsc.load_gather(buf, [row_base + lab16])
        inter = (labval > t).astype(jnp.float32)
        union = cnt + 1.0 - inter
        return acc + inter / union

    acc = lax.fori_loop(0, _GROUPS, group_body,
                        jnp.zeros((_L,), jnp.float32))
    acc_v[...] = acc
    pltpu.sync_copy(acc_v, out_hbm.at[wid])


def kernel(prob, label):
    partials = _iou_partials(prob.reshape(-1), label)
    return jnp.sum(partials) / jnp.float32(_BATCH)


# speedup vs baseline: 9.5904x; 1.2096x over previous
import functools

import jax
import jax.numpy as jnp
from jax import lax
from jax.experimental import pallas as pl
from jax.experimental.pallas import tpu as pltpu
from jax.experimental.pallas import tpu_sc as plsc

_BATCH = 16384
_NCLS = 392
_NC = 2
_NS = 16
_NW = _NC * _NS
_L = 16
_ROWS_PER_W = _BATCH // _NW
_CHUNK = 64
_NCHUNK = _ROWS_PER_W // _CHUNK
_GPC = _CHUNK // _L
_CHUNK_ELEMS = _CHUNK * _NCLS

_mesh = plsc.VectorSubcoreMesh(core_axis_name="c", subcore_axis_name="s")


@functools.partial(
    pl.kernel,
    mesh=_mesh,
    out_type=jax.ShapeDtypeStruct((_NW, _L), jnp.float32),
    scratch_types=[
        pltpu.VMEM((_CHUNK_ELEMS,), jnp.float32),
        pltpu.VMEM((_CHUNK_ELEMS,), jnp.float32),
        pltpu.VMEM((_ROWS_PER_W,), jnp.int32),
        pltpu.VMEM((_L,), jnp.float32),
        pltpu.SemaphoreType.DMA,
        pltpu.SemaphoreType.DMA,
    ],
    compiler_params=pltpu.CompilerParams(
        use_tc_tiling_on_sc=False, needs_layout_passes=False),
)
def _iou_partials(prob_hbm, label_hbm, out_hbm,
                  buf_a, buf_b, lab_v, acc_v, sem_a, sem_b):
    wid = lax.axis_index("s") * _NC + lax.axis_index("c")
    row0 = wid * _ROWS_PER_W
    pltpu.sync_copy(label_hbm.at[pl.ds(row0, _ROWS_PER_W)], lab_v)

    iota = lax.iota(jnp.int32, _L)
    row_base = iota * _NCLS
    neg_inf = jnp.full((_L,), -jnp.inf, jnp.float32)

    def chunk_copy(c, buf, sem):
        return pltpu.make_async_copy(
            prob_hbm.at[pl.ds((row0 + c * _CHUNK) * _NCLS, _CHUNK_ELEMS)],
            buf, sem)

    def process(buf, chunk, acc):
        def group_body(gi, acc):
            gbase = row_base + gi * (_L * _NCLS)

            def j_body(j, ms):
                m0, m1, m2, m3, m4, m5 = ms
                x = plsc.load_gather(buf, [gbase + j])
                n0 = jnp.maximum(m0, x)
                c = jnp.minimum(m0, x)
                n1 = jnp.maximum(m1, c)
                c = jnp.minimum(m1, c)
                n2 = jnp.maximum(m2, c)
                c = jnp.minimum(m2, c)
                n3 = jnp.maximum(m3, c)
                c = jnp.minimum(m3, c)
                n4 = jnp.maximum(m4, c)
                c = jnp.minimum(m4, c)
                n5 = jnp.maximum(m5, c)
                return (n0, n1, n2, n3, n4, n5)

            m0, m1, m2, m3, m4, m5 = lax.fori_loop(
                0, _NCLS, j_body, (neg_inf,) * 6, unroll=8)

            t = m5
            cnt = ((m0 > t).astype(jnp.float32)
                   + (m1 > t).astype(jnp.float32)
                   + (m2 > t).astype(jnp.float32)
                   + (m3 > t).astype(jnp.float32)
                   + (m4 > t).astype(jnp.float32))
            lab16 = plsc.load_gather(
                lab_v, [(chunk * _GPC + gi) * _L + iota])
            labval = plsc.load_gather(buf, [gbase + lab16])
            inter = (labval > t).astype(jnp.float32)
            union = cnt + 1.0 - inter
            return acc + inter / union

        return lax.fori_loop(0, _GPC, group_body, acc)

    chunk_copy(0, buf_a, sem_a).start()

    def pair_body(i, acc):
        ca = 2 * i
        cb = 2 * i + 1
        chunk_copy(ca, buf_a, sem_a).wait()
        chunk_copy(cb, buf_b, sem_b).start()
        acc = process(buf_a, ca, acc)
        chunk_copy(cb, buf_b, sem_b).wait()
        chunk_copy(jnp.minimum(ca + 2, _NCHUNK - 1), buf_a, sem_a).start()
        acc = process(buf_b, cb, acc)
        return acc

    acc = lax.fori_loop(0, _NCHUNK // 2, pair_body,
                        jnp.zeros((_L,), jnp.float32))
    chunk_copy(_NCHUNK - 1, buf_a, sem_a).wait()
    acc_v[...] = acc
    pltpu.sync_copy(acc_v, out_hbm.at[wid])


def kernel(prob, label):
    partials = _iou_partials(prob.reshape(-1), label)
    return jnp.sum(partials) / jnp.float32(_BATCH)
